# baseline (speedup 1.0000x reference)
import functools

import jax
import jax.numpy as jnp
from jax import lax
from jax.experimental import pallas as pl
from jax.experimental.pallas import tpu as pltpu
from jax.experimental.pallas import tpu_sc as plsc

NUM_TASKS = 8
NUM_TOKENS = 4096
INPUT_DIM = 1024
NUM_CLASSES = 512

TB = 512
NB = NUM_TOKENS // TB + NUM_TASKS
P = NB * TB

_NW = 32


def _sc_mesh():
    return plsc.VectorSubcoreMesh(core_axis_name="c", subcore_axis_name="s")


_S_ROWS = NUM_TOKENS // _NW
_S_CH = 32
_S_NCH = _S_ROWS // _S_CH


def _scatter_feat_kernel(feat_hbm, idx_hbm, out_hbm, i0, i1, i2, i3,
                         rows0, rows1, sem0, sem1):
    wid = lax.axis_index("s") * 2 + lax.axis_index("c")
    base = wid * _S_ROWS
    idxs = (i0, i1, i2, i3)
    for c in range(_S_NCH):
        pltpu.sync_copy(idx_hbm.at[pl.ds(base + c * _S_CH, _S_CH)], idxs[c])
    bufs, sems, cps = (rows0, rows1), (sem0, sem1), [None, None]
    for c in range(_S_NCH):
        if c >= 2:
            cps[c % 2].wait()
        pltpu.sync_copy(feat_hbm.at[pl.ds(base + c * _S_CH, _S_CH)],
                        bufs[c % 2])
        cps[c % 2] = pltpu.async_copy(bufs[c % 2], out_hbm.at[idxs[c]],
                                      sems[c % 2])
    cps[0].wait()
    cps[1].wait()


def _scatter_feat(feature, p):
    return pl.kernel(
        _scatter_feat_kernel,
        mesh=_sc_mesh(),
        out_type=jax.ShapeDtypeStruct((P, INPUT_DIM), jnp.float32),
        scratch_types=[
            pltpu.VMEM((_S_CH,), jnp.int32),
            pltpu.VMEM((_S_CH,), jnp.int32),
            pltpu.VMEM((_S_CH,), jnp.int32),
            pltpu.VMEM((_S_CH,), jnp.int32),
            pltpu.VMEM((_S_CH, INPUT_DIM), jnp.float32),
            pltpu.VMEM((_S_CH, INPUT_DIM), jnp.float32),
            pltpu.SemaphoreType.DMA,
            pltpu.SemaphoreType.DMA,
        ],
    )(feature, p)


_C_ROWS = NUM_TOKENS // _NW


def _combine_kernel(slog_hbm, idx_hbm, out_hbm, idx_v, rows_v, sem):
    wid = lax.axis_index("s") * 2 + lax.axis_index("c")
    base = wid * _C_ROWS
    pltpu.sync_copy(idx_hbm.at[pl.ds(base, _C_ROWS)], idx_v)
    pltpu.async_copy(slog_hbm.at[idx_v], rows_v, sem).wait()
    pltpu.sync_copy(rows_v, out_hbm.at[pl.ds(base, _C_ROWS)])


def _combine(sorted_logits, p):
    return pl.kernel(
        _combine_kernel,
        mesh=_sc_mesh(),
        out_type=jax.ShapeDtypeStruct((NUM_TOKENS, NUM_CLASSES), jnp.float32),
        scratch_types=[
            pltpu.VMEM((_C_ROWS,), jnp.int32),
            pltpu.VMEM((_C_ROWS, NUM_CLASSES), jnp.float32),
            pltpu.SemaphoreType.DMA,
        ],
    )(sorted_logits, p)


def _mm_body(nu_ref, bt_ref, x_ref, w_ref, b_ref, o_ref):
    i = pl.program_id(0)

    @pl.when(i < nu_ref[0])
    def _():
        x = x_ref[...]
        w = w_ref[bt_ref[i]]
        y = lax.dot_general(x, w, (((1,), (1,)), ((), ())),
                            preferred_element_type=jnp.float32)
        o_ref[...] = y + b_ref[bt_ref[i], 0][None, :]


def _grouped_matmul(sorted_feat, W, b, nb_used, block_task):
    grid_spec = pltpu.PrefetchScalarGridSpec(
        num_scalar_prefetch=2,
        grid=(NB,),
        in_specs=[
            pl.BlockSpec((TB, INPUT_DIM),
                         lambda i, nu, bt: (jnp.minimum(i, nu[0] - 1), 0)),
            pl.BlockSpec((NUM_TASKS, NUM_CLASSES, INPUT_DIM),
                         lambda i, nu, bt: (0, 0, 0)),
            pl.BlockSpec((NUM_TASKS, 1, NUM_CLASSES),
                         lambda i, nu, bt: (0, 0, 0)),
        ],
        out_specs=pl.BlockSpec((TB, NUM_CLASSES),
                               lambda i, nu, bt: (jnp.minimum(i, nu[0] - 1),
                                                  0)),
    )
    return pl.pallas_call(
        _mm_body,
        grid_spec=grid_spec,
        out_shape=jax.ShapeDtypeStruct((P, NUM_CLASSES), jnp.float32),
    )(nb_used, block_task, sorted_feat, W,
      b.reshape(NUM_TASKS, 1, NUM_CLASSES))


def kernel(feature, task_ids, W, b):
    t = task_ids.astype(jnp.int32)
    onehot = (t[:, None] == jnp.arange(NUM_TASKS, dtype=jnp.int32)[None, :]
              ).astype(jnp.int32)
    csum = jnp.cumsum(onehot, axis=0)
    rank = jnp.sum(onehot * (csum - 1), axis=1)
    counts = csum[-1]
    blocks_per = (counts + TB - 1) // TB
    cumb = jnp.cumsum(blocks_per)
    padded_off = TB * (cumb - blocks_per)
    p = (jnp.sum(onehot * padded_off[None, :], axis=1)
         + rank).astype(jnp.int32)
    nb_used = cumb[-1:].astype(jnp.int32)
    block_task = jnp.minimum(
        jnp.sum((cumb[None, :] <= jnp.arange(NB, dtype=jnp.int32)[:, None]
                 ).astype(jnp.int32), axis=1),
        NUM_TASKS - 1).astype(jnp.int32)

    sorted_feat = _scatter_feat(feature, p)
    sorted_logits = _grouped_matmul(sorted_feat, W, b, nb_used, block_task)
    return _combine(sorted_logits, p)


# speedup vs baseline: 1.0860x; 1.0860x over previous
import functools

import jax
import jax.numpy as jnp
from jax import lax
from jax.experimental import pallas as pl
from jax.experimental.pallas import tpu as pltpu
from jax.experimental.pallas import tpu_sc as plsc

NUM_TASKS = 8
NUM_TOKENS = 4096
INPUT_DIM = 1024
NUM_CLASSES = 512

TB = 512
NB = NUM_TOKENS // TB + NUM_TASKS
P = NB * TB

_NW = 32


def _sc_mesh():
    return plsc.VectorSubcoreMesh(core_axis_name="c", subcore_axis_name="s")


_S_ROWS = NUM_TOKENS // _NW
_S_CH = 32
_S_NCH = _S_ROWS // _S_CH


def _scatter_feat_kernel(feat_hbm, idx_hbm, out_hbm, i0, i1, i2, i3,
                         rows0, rows1, sem0, sem1):
    wid = lax.axis_index("s") * 2 + lax.axis_index("c")
    base = wid * _S_ROWS
    idxs = (i0, i1, i2, i3)
    for c in range(_S_NCH):
        pltpu.sync_copy(idx_hbm.at[pl.ds(base + c * _S_CH, _S_CH)], idxs[c])
    bufs, sems, cps = (rows0, rows1), (sem0, sem1), [None, None]
    for c in range(_S_NCH):
        if c >= 2:
            cps[c % 2].wait()
        pltpu.sync_copy(feat_hbm.at[pl.ds(base + c * _S_CH, _S_CH)],
                        bufs[c % 2])
        cps[c % 2] = pltpu.async_copy(bufs[c % 2], out_hbm.at[idxs[c]],
                                      sems[c % 2])
    cps[0].wait()
    cps[1].wait()


def _scatter_feat(feature, p):
    return pl.kernel(
        _scatter_feat_kernel,
        mesh=_sc_mesh(),
        out_type=jax.ShapeDtypeStruct((P, INPUT_DIM), jnp.float32),
        scratch_types=[
            pltpu.VMEM((_S_CH,), jnp.int32),
            pltpu.VMEM((_S_CH,), jnp.int32),
            pltpu.VMEM((_S_CH,), jnp.int32),
            pltpu.VMEM((_S_CH,), jnp.int32),
            pltpu.VMEM((_S_CH, INPUT_DIM), jnp.float32),
            pltpu.VMEM((_S_CH, INPUT_DIM), jnp.float32),
            pltpu.SemaphoreType.DMA,
            pltpu.SemaphoreType.DMA,
        ],
    )(feature, p)


_C_ROWS = NUM_TOKENS // _NW


def _combine_kernel(slog_hbm, idx_hbm, out_hbm, idx_v, rows_v, sem):
    wid = lax.axis_index("s") * 2 + lax.axis_index("c")
    base = wid * _C_ROWS
    pltpu.sync_copy(idx_hbm.at[pl.ds(base, _C_ROWS)], idx_v)
    pltpu.async_copy(slog_hbm.at[idx_v], rows_v, sem).wait()
    pltpu.sync_copy(rows_v, out_hbm.at[pl.ds(base, _C_ROWS)])


def _combine(sorted_logits, p):
    return pl.kernel(
        _combine_kernel,
        mesh=_sc_mesh(),
        out_type=jax.ShapeDtypeStruct((NUM_TOKENS, NUM_CLASSES), jnp.float32),
        scratch_types=[
            pltpu.VMEM((_C_ROWS,), jnp.int32),
            pltpu.VMEM((_C_ROWS, NUM_CLASSES), jnp.float32),
            pltpu.SemaphoreType.DMA,
        ],
    )(sorted_logits, p)


_M_R = 32
_M_C = NUM_TOKENS // _M_R


def _meta_body(t_ref, p_ref, nb_ref, bt_ref):
    t = t_ref[...]
    iota_nb = lax.broadcasted_iota(jnp.int32, (1, NB), 1)
    tri_c = (lax.broadcasted_iota(jnp.int32, (_M_C, _M_C), 0)
             <= lax.broadcasted_iota(jnp.int32, (_M_C, _M_C), 1)
             ).astype(jnp.float32)
    tri_r = (lax.broadcasted_iota(jnp.int32, (_M_R, _M_R), 1)
             < lax.broadcasted_iota(jnp.int32, (_M_R, _M_R), 0)
             ).astype(jnp.float32)
    p = jnp.zeros((_M_R, _M_C), jnp.int32)
    btacc = jnp.zeros((1, NB), jnp.int32)
    cumb = jnp.int32(0)
    for e in range(NUM_TASKS):
        m = t == e
        mf = m.astype(jnp.float32)
        row_cum = lax.dot_general(mf, tri_c, (((1,), (0,)), ((), ())),
                                  preferred_element_type=jnp.float32)
        row_tot = row_cum[:, -1:]
        pre = lax.dot_general(tri_r, row_tot, (((1,), (0,)), ((), ())),
                              preferred_element_type=jnp.float32)
        rank = (pre + row_cum - mf).astype(jnp.int32)
        cnt = jnp.sum(mf).astype(jnp.int32)
        p = jnp.where(m, TB * cumb + rank, p)
        cumb = cumb + (cnt + TB - 1) // TB
        btacc = btacc + (cumb <= iota_nb).astype(jnp.int32)
    p_ref[...] = p
    nb_ref[...] = jnp.full((1, 1), cumb, jnp.int32)
    bt_ref[...] = jnp.minimum(btacc, NUM_TASKS - 1)


def _metadata(task_ids):
    p2d, nb, bt = pl.pallas_call(
        _meta_body,
        out_shape=[
            jax.ShapeDtypeStruct((_M_R, _M_C), jnp.int32),
            jax.ShapeDtypeStruct((1, 1), jnp.int32),
            jax.ShapeDtypeStruct((1, NB), jnp.int32),
        ],
    )(task_ids.astype(jnp.int32).reshape(_M_R, _M_C))
    return p2d.reshape(NUM_TOKENS), nb.reshape(1), bt.reshape(NB)


def _mm_body(nu_ref, bt_ref, x_ref, w_ref, b_ref, o_ref):
    i = pl.program_id(0)

    @pl.when(i < nu_ref[0])
    def _():
        x = x_ref[...]
        w = w_ref[0]
        y = lax.dot_general(x, w, (((1,), (1,)), ((), ())),
                            preferred_element_type=jnp.float32)
        o_ref[...] = y + b_ref[0, 0][None, :]


def _grouped_matmul(sorted_feat, W, b, nb_used, block_task):
    grid_spec = pltpu.PrefetchScalarGridSpec(
        num_scalar_prefetch=2,
        grid=(NB,),
        in_specs=[
            pl.BlockSpec((TB, INPUT_DIM),
                         lambda i, nu, bt: (jnp.minimum(i, nu[0] - 1), 0)),
            pl.BlockSpec((1, NUM_CLASSES, INPUT_DIM),
                         lambda i, nu, bt: (bt[i], 0, 0)),
            pl.BlockSpec((1, 1, NUM_CLASSES), lambda i, nu, bt: (bt[i], 0, 0)),
        ],
        out_specs=pl.BlockSpec((TB, NUM_CLASSES),
                               lambda i, nu, bt: (jnp.minimum(i, nu[0] - 1),
                                                  0)),
    )
    return pl.pallas_call(
        _mm_body,
        grid_spec=grid_spec,
        out_shape=jax.ShapeDtypeStruct((P, NUM_CLASSES), jnp.float32),
    )(nb_used, block_task, sorted_feat, W,
      b.reshape(NUM_TASKS, 1, NUM_CLASSES))


def kernel(feature, task_ids, W, b):
    p, nb_used, block_task = _metadata(task_ids)
    sorted_feat = _scatter_feat(feature, p)
    sorted_logits = _grouped_matmul(sorted_feat, W, b, nb_used, block_task)
    return _combine(sorted_logits, p)
